# baseline (speedup 1.0000x reference)
import dataclasses
import functools

import jax
import jax.numpy as jnp
from jax import lax
from jax.experimental import pallas as pl
from jax.experimental.pallas import tpu as pltpu
from jax.experimental.pallas import tpu_sc as plsc

N = 10000
E = 640000
HEADS = 4
HID = 128

NP = 10240
N16 = NP * 16
NCORES = 2
NSUB = 16
NW = NCORES * NSUB
EPW = E // NW
CH1 = 160
CH2 = 64
NCH2 = (EPW + CH2 - 1) // CH2
NIT2 = NCH2 + 1
EPAD = 192
E4PAD = 1024

_SC_MESH = plsc.VectorSubcoreMesh(core_axis_name="c", subcore_axis_name="s")

_SC_PARAMS = pltpu.CompilerParams()
if "needs_layout_passes" in pltpu.CompilerParams.__dataclass_fields__:
    _SC_PARAMS = dataclasses.replace(_SC_PARAMS, needs_layout_passes=False)


def _dense_stage_kernel(x_ref, wsrc_ref, wdst_ref, asrc_ref, adst_ref,
                        wlin_ref, blin_ref,
                        xs_ref, as_ref, ad_ref, lin_ref, ms_ref, md_ref):
    i = pl.program_id(0)
    x = x_ref[...]
    xs = jax.lax.dot_general(x, wsrc_ref[...], (((1,), (0,)), ((), ())),
                             preferred_element_type=jnp.float32,
                             precision=jax.lax.Precision.HIGHEST)
    bn = xs.shape[0]
    xs_ref[...] = xs.astype(jnp.bfloat16)
    att_s = asrc_ref[...].reshape(1, HEADS * HID)
    a_s = (xs * att_s).reshape(bn, HEADS, HID).sum(-1)
    as_ref[...] = a_s
    xd = jax.lax.dot_general(x, wdst_ref[...], (((1,), (0,)), ((), ())),
                             preferred_element_type=jnp.float32,
                             precision=jax.lax.Precision.HIGHEST)
    att_d = adst_ref[...].reshape(1, HEADS * HID)
    a_d = (xd * att_d).reshape(bn, HEADS, HID).sum(-1)
    ad_ref[...] = a_d
    lin_ref[...] = jax.lax.dot_general(
        x, wlin_ref[...], (((1,), (0,)), ((), ())),
        preferred_element_type=jnp.float32,
        precision=jax.lax.Precision.HIGHEST) + blin_ref[...].reshape(1, HID)
    bms = a_s.max(axis=0, keepdims=True)
    bmd = a_d.max(axis=0, keepdims=True)

    @pl.when(i == 0)
    def _():
        ms_ref[...] = bms
        md_ref[...] = bmd

    @pl.when(i > 0)
    def _():
        ms_ref[...] = jnp.maximum(ms_ref[...], bms)
        md_ref[...] = jnp.maximum(md_ref[...], bmd)


def _dense_stage(x, W_src, W_dst, att_src, att_dst, W_lin, b_lin):
    n, din = x.shape
    bn = 2000
    grid = (n // bn,)
    return pl.pallas_call(
        _dense_stage_kernel,
        grid=grid,
        in_specs=[
            pl.BlockSpec((bn, din), lambda i: (i, 0)),
            pl.BlockSpec((din, HEADS * HID), lambda i: (0, 0)),
            pl.BlockSpec((din, HEADS * HID), lambda i: (0, 0)),
            pl.BlockSpec((1, HEADS, HID), lambda i: (0, 0, 0)),
            pl.BlockSpec((1, HEADS, HID), lambda i: (0, 0, 0)),
            pl.BlockSpec((din, HID), lambda i: (0, 0)),
            pl.BlockSpec((HID,), lambda i: (0,)),
        ],
        out_specs=[
            pl.BlockSpec((bn, HEADS * HID), lambda i: (i, 0)),
            pl.BlockSpec((bn, HEADS), lambda i: (i, 0)),
            pl.BlockSpec((bn, HEADS), lambda i: (i, 0)),
            pl.BlockSpec((bn, HID), lambda i: (i, 0)),
            pl.BlockSpec((1, HEADS), lambda i: (0, 0)),
            pl.BlockSpec((1, HEADS), lambda i: (0, 0)),
        ],
        out_shape=[
            jax.ShapeDtypeStruct((n, HEADS * HID), jnp.bfloat16),
            jax.ShapeDtypeStruct((n, HEADS), jnp.float32),
            jax.ShapeDtypeStruct((n, HEADS), jnp.float32),
            jax.ShapeDtypeStruct((n, HID), jnp.float32),
            jax.ShapeDtypeStruct((1, HEADS), jnp.float32),
            jax.ShapeDtypeStruct((1, HEADS), jnp.float32),
        ],
    )(x, W_src, W_dst, att_src, att_dst, W_lin, b_lin)


@functools.partial(
    pl.kernel,
    mesh=_SC_MESH,
    out_type=[
        jax.ShapeDtypeStruct((E * HEADS + E4PAD,), jnp.float32),
        jax.ShapeDtypeStruct((NCORES, NP * HEADS), jnp.float32),
    ],
    scratch_types=[
        pltpu.VMEM((NP * HEADS,), jnp.float32),
        pltpu.VMEM((NP * HEADS,), jnp.float32),
        pltpu.VMEM((16,), jnp.float32),
        pltpu.VMEM((CH1,), jnp.int32),
        pltpu.VMEM((CH1,), jnp.int32),
        pltpu.VMEM((CH1,), jnp.int32),
        pltpu.VMEM((CH1,), jnp.int32),
        pltpu.VMEM((CH1 * HEADS,), jnp.float32),
        pltpu.VMEM((CH1 * HEADS,), jnp.float32),
        pltpu.VMEM((128,), jnp.int32),
        pltpu.VMEM((2560,), jnp.float32),
        pltpu.VMEM_SHARED((NP * HEADS,), jnp.float32),
        pltpu.SemaphoreType.DMA,
        pltpu.SemaphoreType.DMA,
        pltpu.SemaphoreType.DMA,
        pltpu.SemaphoreType.DMA,
    ],
    compiler_params=_SC_PARAMS,
)
def _sc_pass1(asrc_hbm, adst_hbm, l_hbm, src_hbm, dst_hbm,
              ex_hbm, denom_hbm,
              asrc_v, adst_v, l_v, src_a, src_b, dst_a, dst_b, ex_a, ex_b,
              idx_v, z_v, denom_sh, si_a, si_b, sw_a, sw_b):
    ci = lax.axis_index("c")
    si = lax.axis_index("s")
    wid = si * NCORES + ci

    src = (src_a, src_b)
    dst = (dst_a, dst_b)
    ex = (ex_a, ex_b)
    sin = (si_a, si_b)
    swb = (sw_a, sw_b)

    NCH1 = EPW // CH1

    @pl.loop(0, 2560, step=16)
    def _(i):
        z_v[pl.ds(i, 16)] = jnp.zeros((16,), jnp.float32)

    pltpu.sync_copy(z_v, denom_sh.at[pl.ds(si * 2560, 2560)])

    pltpu.sync_copy(asrc_hbm, asrc_v)
    pltpu.sync_copy(adst_hbm, adst_v)
    pltpu.sync_copy(l_hbm, l_v)
    plsc.subcore_barrier()

    lane = lax.iota(jnp.int32, 16)
    h_pat = lane & 3
    e_pat = lane >> 2
    lvec = l_v[...]

    def chunk_base(k):
        return wid * EPW + jnp.minimum(k, NCH1) * CH1

    def in_descs(k, b):
        base = chunk_base(k)
        return (
            pltpu.make_async_copy(src_hbm.at[pl.ds(base, CH1)],
                                  src[b], sin[b]),
            pltpu.make_async_copy(dst_hbm.at[pl.ds(base, CH1)],
                                  dst[b], sin[b]),
        )

    def wb_desc(k, b):
        return pltpu.make_async_copy(
            ex[b], ex_hbm.at[pl.ds(chunk_base(k) * 4, CH1 * HEADS)], swb[b])

    for d in in_descs(0, 0):
        d.start()
    for d in in_descs(1, 1):
        d.start()

    @pl.loop(0, NCH1 + 1, step=2)
    def _(k0):
        for b in (0, 1):
            k = k0 + b
            for d in in_descs(k, b):
                d.wait()

            @pl.when(k >= 2)
            def _():
                wb_desc(k - 2, b).wait()

            @plsc.parallel_loop(0, CH1 * HEADS // 16, unroll=2)
            def _(g):
                e_vec = g * 4 + e_pat
                s_idx = plsc.load_gather(src[b], [e_vec])
                d_idx = plsc.load_gather(dst[b], [e_vec])
                a_s = plsc.load_gather(asrc_v, [s_idx * 4 + h_pat])
                a_d = plsc.load_gather(adst_v, [d_idx * 4 + h_pat])
                al = a_s + a_d
                al = jnp.maximum(al, al * 0.2)
                ex[b][pl.ds(g * 16, 16)] = jnp.exp(al - lvec)

            wb_desc(k, b).start()

            @pl.when(k < NCH1)
            def _():
                @pl.loop(0, CH1 * HEADS // 128)
                def _(s):
                    @plsc.parallel_loop(0, 8, unroll=2)
                    def _(g):
                        e_vec = s * 32 + g * 4 + e_pat
                        d_idx = plsc.load_gather(dst[b], [e_vec])
                        idx_v[pl.ds(g * 16, 16)] = d_idx * 4 + h_pat

                    pltpu.sync_copy(ex[b].at[pl.ds(s * 128, 128)],
                                    denom_sh.at[idx_v], add=True)

            for d in in_descs(k + 2, b):
                d.start()

    for d in in_descs(NCH1 + 2, 0):
        d.wait()
    for d in in_descs(NCH1 + 3, 1):
        d.wait()
    wb_desc(NCH1 - 1, 0).wait()
    wb_desc(NCH1, 1).wait()

    plsc.subcore_barrier()
    pltpu.sync_copy(denom_sh.at[pl.ds(si * 2560, 2560)],
                    denom_hbm.at[ci, pl.ds(si * 2560, 2560)])


def _inv_kernel(d_ref, inv_ref):
    d = d_ref[...]
    inv_ref[...] = 0.25 / (d[0] + d[1] + 1e-16)


def _inv_denom(denom):
    d3 = denom.reshape(NCORES, NP * HEADS // 128, 128)
    out = pl.pallas_call(
        _inv_kernel,
        out_shape=jax.ShapeDtypeStruct((NP * HEADS // 128, 128), jnp.float32),
    )(d3)
    return out.reshape(NP * HEADS)


@functools.partial(
    pl.kernel,
    mesh=_SC_MESH,
    out_type=jax.ShapeDtypeStruct((E * HEADS + E4PAD,), jnp.float32),
    scratch_types=[
        pltpu.VMEM((NP * HEADS,), jnp.float32),
        pltpu.VMEM((CH1,), jnp.int32),
        pltpu.VMEM((CH1,), jnp.int32),
        pltpu.VMEM((CH1 * HEADS,), jnp.float32),
        pltpu.VMEM((CH1 * HEADS,), jnp.float32),
        pltpu.SemaphoreType.DMA,
        pltpu.SemaphoreType.DMA,
        pltpu.SemaphoreType.DMA,
        pltpu.SemaphoreType.DMA,
    ],
    compiler_params=_SC_PARAMS,
)
def _sc_coef(invd_hbm, ex_hbm, dst_hbm, coef_hbm,
             inv_v, dst_a, dst_b, ex_a, ex_b, si_a, si_b, sw_a, sw_b):
    ci = lax.axis_index("c")
    si = lax.axis_index("s")
    wid = si * NCORES + ci

    dst = (dst_a, dst_b)
    ex = (ex_a, ex_b)
    sin = (si_a, si_b)
    swb = (sw_a, sw_b)

    NCH1 = EPW // CH1

    pltpu.sync_copy(invd_hbm, inv_v)

    lane = lax.iota(jnp.int32, 16)
    h_pat = lane & 3
    e_pat = lane >> 2

    def chunk_base(k):
        return wid * EPW + jnp.minimum(k, NCH1) * CH1

    def in_descs(k, b):
        base = chunk_base(k)
        return (
            pltpu.make_async_copy(dst_hbm.at[pl.ds(base, CH1)],
                                  dst[b], sin[b]),
            pltpu.make_async_copy(ex_hbm.at[pl.ds(base * 4, CH1 * HEADS)],
                                  ex[b], sin[b]),
        )

    def wb_desc(k, b):
        return pltpu.make_async_copy(
            ex[b], coef_hbm.at[pl.ds(chunk_base(k) * 4, CH1 * HEADS)], swb[b])

    for d in in_descs(0, 0):
        d.start()
    for d in in_descs(1, 1):
        d.start()

    @pl.loop(0, NCH1 + 1, step=2)
    def _(k0):
        for b in (0, 1):
            k = k0 + b
            for d in in_descs(k, b):
                d.wait()

            @pl.when(k >= 2)
            def _():
                wb_desc(k - 2, b).wait()

            @plsc.parallel_loop(0, CH1 * HEADS // 16, unroll=2)
            def _(g):
                e_vec = g * 4 + e_pat
                d_idx = plsc.load_gather(dst[b], [e_vec])
                iv = plsc.load_gather(inv_v, [d_idx * 4 + h_pat])
                ex[b][pl.ds(g * 16, 16)] = ex[b][pl.ds(g * 16, 16)] * iv

            wb_desc(k, b).start()
            for d in in_descs(k + 2, b):
                d.start()

    for d in in_descs(NCH1 + 2, 0):
        d.wait()
    for d in in_descs(NCH1 + 3, 1):
        d.wait()
    wb_desc(NCH1 - 1, 0).wait()
    wb_desc(NCH1, 1).wait()


@functools.partial(
    pl.kernel,
    mesh=_SC_MESH,
    out_type=jax.ShapeDtypeStruct((NCORES, NP, HID), jnp.float32),
    scratch_types=[
        pltpu.VMEM((CH2, HEADS * HID // 2), jnp.int32),
        pltpu.VMEM((CH2, HEADS * HID // 2), jnp.int32),
        pltpu.VMEM((CH2, HID), jnp.float32),
        pltpu.VMEM((CH2 * HEADS,), jnp.float32),
        pltpu.VMEM((CH2 * HEADS,), jnp.float32),
        pltpu.VMEM((CH2,), jnp.int32),
        pltpu.VMEM((CH2,), jnp.int32),
        pltpu.VMEM((CH2,), jnp.int32),
        pltpu.VMEM((CH2,), jnp.int32),
        pltpu.VMEM((CH2,), jnp.int32),
        pltpu.VMEM_SHARED((NP, HID), jnp.float32),
        pltpu.SemaphoreType.DMA,
        pltpu.SemaphoreType.DMA,
        pltpu.SemaphoreType.DMA,
        pltpu.SemaphoreType.DMA,
        pltpu.SemaphoreType.DMA,
    ],
    compiler_params=_SC_PARAMS,
)
def _sc_pass2(xs_hbm, coef_hbm, src_hbm, dst_hbm,
              out_hbm,
              rows_a, rows_b, m_v, coef_a, coef_b, src_a, src_b,
              dst_a, dst_b, dsc_v,
              out_sh, sg_a, sg_b, s1_a, s1_b, sc_v):
    ci = lax.axis_index("c")
    si = lax.axis_index("s")
    wid = si * NCORES + ci

    rows = (rows_a, rows_b)
    coef = (coef_a, coef_b)
    src = (src_a, src_b)
    dst = (dst_a, dst_b)
    sg = (sg_a, sg_b)
    s1 = (s1_a, s1_b)

    @pl.loop(0, CH2)
    def _(r):
        for c in range(HID // 16):
            m_v[r, pl.ds(c * 16, 16)] = jnp.zeros((16,), jnp.float32)

    @pl.loop(0, 640 // CH2)
    def _(j):
        pltpu.sync_copy(m_v, out_sh.at[pl.ds(si * 640 + j * CH2, CH2)])

    plsc.subcore_barrier()

    def stage1_descs(k, b):
        base = wid * EPW + jnp.minimum(k, NCH2) * CH2
        return (
            pltpu.make_async_copy(src_hbm.at[pl.ds(base, CH2)], src[b], s1[b]),
            pltpu.make_async_copy(dst_hbm.at[pl.ds(base, CH2)], dst[b], s1[b]),
            pltpu.make_async_copy(coef_hbm.at[pl.ds(base * 4, CH2 * HEADS)],
                                  coef[b], s1[b]),
        )

    def issue_stage1(k, b):
        for d in stage1_descs(k, b):
            d.start()

    def wait_stage1(k, b):
        for d in stage1_descs(k, b):
            d.wait()

    issue_stage1(0, 0)
    issue_stage1(1, 1)
    wait_stage1(0, 0)
    pltpu.make_async_copy(xs_hbm.at[src[0]], rows[0], sg[0]).start()

    @pl.loop(0, NIT2, step=2)
    def _(k0):
        for b in (0, 1):
            ob = 1 - b
            k = k0 + b
            pltpu.make_async_copy(xs_hbm.at[src[b]], rows[b], sg[b]).wait()
            wait_stage1(k + 1, ob)
            pltpu.make_async_copy(xs_hbm.at[src[ob]], rows[ob], sg[ob]).start()

            @pl.when(k >= 1)
            def _():
                pltpu.make_async_copy(m_v, out_sh.at[dsc_v], sc_v).wait()

            for j in range(CH2 // 16):
                dsc_v[pl.ds(j * 16, 16)] = dst[b][pl.ds(j * 16, 16)]

            mask0 = jnp.clip((EPW - k * CH2) * HEADS, 0, CH2 * HEADS)

            @pl.loop(mask0, CH2 * HEADS, step=16)
            def _(p):
                coef[b][pl.ds(p, 16)] = jnp.zeros((16,), jnp.float32)

            @plsc.parallel_loop(0, CH2, unroll=4)
            def _(e):
                cfs = [plsc.load_gather(
                           coef[b],
                           [jnp.full((16,), e * HEADS + h, jnp.int32)])
                       for h in range(HEADS)]
                for c in range(HID // 32):
                    acc_lo = None
                    acc_hi = None
                    for h in range(HEADS):
                        xi = rows[b][e, pl.ds(h * (HID // 2) + c * 16, 16)]
                        x32 = plsc.bitcast(xi, jnp.bfloat16)
                        lo, hi = plsc.unpack(
                            x32, format=plsc.PackFormat.INTERLEAVED)
                        if acc_lo is None:
                            acc_lo = lo * cfs[h]
                            acc_hi = hi * cfs[h]
                        else:
                            acc_lo = acc_lo + lo * cfs[h]
                            acc_hi = acc_hi + hi * cfs[h]
                    m_v[e, pl.ds(c * 32, 16)] = acc_lo
                    m_v[e, pl.ds(c * 32 + 16, 16)] = acc_hi

            pltpu.async_copy(m_v, out_sh.at[dsc_v], sc_v, add=True)
            issue_stage1(k + 2, b)

    pltpu.make_async_copy(xs_hbm.at[src[0]], rows[0], sg[0]).wait()
    wait_stage1(NIT2 + 1, 1)
    pltpu.make_async_copy(m_v, out_sh.at[dsc_v], sc_v).wait()

    plsc.subcore_barrier()
    pltpu.sync_copy(out_sh.at[pl.ds(si * 640, 640)],
                    out_hbm.at[ci, pl.ds(si * 640, 640)])


def _epilogue_kernel(o_ref, lin_ref, bc_ref, h_ref):
    o = o_ref[...]
    h_ref[...] = jax.nn.relu(o[0] + o[1] + bc_ref[...].reshape(1, HID)
                             + lin_ref[...])


def _epilogue(o_parts, lin, b_conv):
    bn = 2000
    return pl.pallas_call(
        _epilogue_kernel,
        grid=(N // bn,),
        in_specs=[
            pl.BlockSpec((NCORES, bn, HID), lambda i: (0, i, 0)),
            pl.BlockSpec((bn, HID), lambda i: (i, 0)),
            pl.BlockSpec((HID,), lambda i: (0,)),
        ],
        out_specs=pl.BlockSpec((bn, HID), lambda i: (i, 0)),
        out_shape=jax.ShapeDtypeStruct((N, HID), jnp.float32),
    )(o_parts, lin, b_conv)


def _gat_layer(h, src, dst, W_src, W_dst, att_src, att_dst, b_conv,
               W_lin, b_lin):
    xs, a_s, a_d, lin, ms, md = _dense_stage(
        h, W_src, W_dst, att_src, att_dst, W_lin, b_lin)
    l4 = jax.nn.leaky_relu(ms + md, negative_slope=0.2)
    l16 = jnp.tile(l4, (1, 4)).reshape(16)
    asrc_f = jnp.pad(a_s.reshape(-1), (0, (NP - N) * HEADS))
    adst_f = jnp.pad(a_d.reshape(-1), (0, (NP - N) * HEADS))
    ex, denom = _sc_pass1(asrc_f, adst_f, l16, src, dst)
    invd = _inv_denom(denom)
    coef = _sc_coef(invd, ex, dst)
    xsp = xs.reshape(N, 16, 2, 16).swapaxes(2, 3).reshape(N, 16, 16, 2)
    xs_i32 = jax.lax.bitcast_convert_type(xsp, jnp.int32).reshape(
        N, HEADS * HID // 2)
    o_parts = _sc_pass2(xs_i32, coef, src, dst)
    return _epilogue(o_parts, lin, b_conv)


def kernel(x, edge_index, W_src1, W_dst1, att_src1, att_dst1, b_conv1, W_lin1, b_lin1, W_src2, W_dst2, att_src2, att_dst2, b_conv2, W_lin2, b_lin2, W_src3, W_dst3, att_src3, att_dst3, b_conv3, W_lin3, b_lin3):
    src = jnp.pad(edge_index[0], (0, EPAD))
    dst = jnp.pad(edge_index[1], (0, EPAD))
    h = _gat_layer(x, src, dst, W_src1, W_dst1, att_src1, att_dst1, b_conv1,
                   W_lin1, b_lin1)
    h = _gat_layer(h, src, dst, W_src2, W_dst2, att_src2, att_dst2, b_conv2,
                   W_lin2, b_lin2)
    h = _gat_layer(h, src, dst, W_src3, W_dst3, att_src3, att_dst3, b_conv3,
                   W_lin3, b_lin3)
    return h


# speedup vs baseline: 1.0078x; 1.0078x over previous
import dataclasses
import functools

import jax
import jax.numpy as jnp
from jax import lax
from jax.experimental import pallas as pl
from jax.experimental.pallas import tpu as pltpu
from jax.experimental.pallas import tpu_sc as plsc

N = 10000
E = 640000
HEADS = 4
HID = 128

NP = 10240
N16 = NP * 16
NCORES = 2
NSUB = 16
NW = NCORES * NSUB
EPW = E // NW
CH1 = 160
CH2 = 64
NCH2 = (EPW + CH2 - 1) // CH2
NIT2 = NCH2 + 1
EPAD = 192
E4PAD = 1024

_SC_MESH = plsc.VectorSubcoreMesh(core_axis_name="c", subcore_axis_name="s")

_SC_PARAMS = pltpu.CompilerParams()
if "needs_layout_passes" in pltpu.CompilerParams.__dataclass_fields__:
    _SC_PARAMS = dataclasses.replace(_SC_PARAMS, needs_layout_passes=False)


def _dense_stage_kernel(x_ref, wsrc_ref, wdst_ref, asrc_ref, adst_ref,
                        wlin_ref, blin_ref,
                        xs_ref, as_ref, ad_ref, lin_ref, ms_ref, md_ref):
    i = pl.program_id(0)
    x = x_ref[...]
    xs = jax.lax.dot_general(x, wsrc_ref[...], (((1,), (0,)), ((), ())),
                             preferred_element_type=jnp.float32,
                             precision=jax.lax.Precision.HIGHEST)
    bn = xs.shape[0]
    xs_ref[...] = xs.astype(jnp.bfloat16)
    att_s = asrc_ref[...].reshape(1, HEADS * HID)
    a_s = (xs * att_s).reshape(bn, HEADS, HID).sum(-1)
    as_ref[...] = a_s
    xd = jax.lax.dot_general(x, wdst_ref[...], (((1,), (0,)), ((), ())),
                             preferred_element_type=jnp.float32,
                             precision=jax.lax.Precision.HIGHEST)
    att_d = adst_ref[...].reshape(1, HEADS * HID)
    a_d = (xd * att_d).reshape(bn, HEADS, HID).sum(-1)
    ad_ref[...] = a_d
    lin_ref[...] = jax.lax.dot_general(
        x, wlin_ref[...], (((1,), (0,)), ((), ())),
        preferred_element_type=jnp.float32,
        precision=jax.lax.Precision.HIGHEST) + blin_ref[...].reshape(1, HID)
    bms = a_s.max(axis=0, keepdims=True)
    bmd = a_d.max(axis=0, keepdims=True)

    @pl.when(i == 0)
    def _():
        ms_ref[...] = bms
        md_ref[...] = bmd

    @pl.when(i > 0)
    def _():
        ms_ref[...] = jnp.maximum(ms_ref[...], bms)
        md_ref[...] = jnp.maximum(md_ref[...], bmd)


def _dense_stage(x, W_src, W_dst, att_src, att_dst, W_lin, b_lin):
    n, din = x.shape
    bn = 2000
    grid = (n // bn,)
    return pl.pallas_call(
        _dense_stage_kernel,
        grid=grid,
        in_specs=[
            pl.BlockSpec((bn, din), lambda i: (i, 0)),
            pl.BlockSpec((din, HEADS * HID), lambda i: (0, 0)),
            pl.BlockSpec((din, HEADS * HID), lambda i: (0, 0)),
            pl.BlockSpec((1, HEADS, HID), lambda i: (0, 0, 0)),
            pl.BlockSpec((1, HEADS, HID), lambda i: (0, 0, 0)),
            pl.BlockSpec((din, HID), lambda i: (0, 0)),
            pl.BlockSpec((HID,), lambda i: (0,)),
        ],
        out_specs=[
            pl.BlockSpec((bn, HEADS * HID), lambda i: (i, 0)),
            pl.BlockSpec((bn, HEADS), lambda i: (i, 0)),
            pl.BlockSpec((bn, HEADS), lambda i: (i, 0)),
            pl.BlockSpec((bn, HID), lambda i: (i, 0)),
            pl.BlockSpec((1, HEADS), lambda i: (0, 0)),
            pl.BlockSpec((1, HEADS), lambda i: (0, 0)),
        ],
        out_shape=[
            jax.ShapeDtypeStruct((n, HEADS * HID), jnp.bfloat16),
            jax.ShapeDtypeStruct((n, HEADS), jnp.float32),
            jax.ShapeDtypeStruct((n, HEADS), jnp.float32),
            jax.ShapeDtypeStruct((n, HID), jnp.float32),
            jax.ShapeDtypeStruct((1, HEADS), jnp.float32),
            jax.ShapeDtypeStruct((1, HEADS), jnp.float32),
        ],
    )(x, W_src, W_dst, att_src, att_dst, W_lin, b_lin)


@functools.partial(
    pl.kernel,
    mesh=_SC_MESH,
    out_type=[
        jax.ShapeDtypeStruct((E * HEADS + E4PAD,), jnp.float32),
        jax.ShapeDtypeStruct((NCORES, NP * HEADS), jnp.float32),
    ],
    scratch_types=[
        pltpu.VMEM((NP * HEADS,), jnp.float32),
        pltpu.VMEM((NP * HEADS,), jnp.float32),
        pltpu.VMEM((16,), jnp.float32),
        pltpu.VMEM((CH1,), jnp.int32),
        pltpu.VMEM((CH1,), jnp.int32),
        pltpu.VMEM((CH1,), jnp.int32),
        pltpu.VMEM((CH1,), jnp.int32),
        pltpu.VMEM((CH1 * HEADS,), jnp.float32),
        pltpu.VMEM((CH1 * HEADS,), jnp.float32),
        pltpu.VMEM((128,), jnp.int32),
        pltpu.VMEM((2560,), jnp.float32),
        pltpu.VMEM_SHARED((NP * HEADS,), jnp.float32),
        pltpu.SemaphoreType.DMA,
        pltpu.SemaphoreType.DMA,
        pltpu.SemaphoreType.DMA,
        pltpu.SemaphoreType.DMA,
    ],
    compiler_params=_SC_PARAMS,
)
def _sc_pass1(asrc_hbm, adst_hbm, l_hbm, src_hbm, dst_hbm,
              ex_hbm, denom_hbm,
              asrc_v, adst_v, l_v, src_a, src_b, dst_a, dst_b, ex_a, ex_b,
              idx_v, z_v, denom_sh, si_a, si_b, sw_a, sw_b):
    ci = lax.axis_index("c")
    si = lax.axis_index("s")
    wid = si * NCORES + ci

    src = (src_a, src_b)
    dst = (dst_a, dst_b)
    ex = (ex_a, ex_b)
    sin = (si_a, si_b)
    swb = (sw_a, sw_b)

    NCH1 = EPW // CH1

    @pl.loop(0, 2560, step=16)
    def _(i):
        z_v[pl.ds(i, 16)] = jnp.zeros((16,), jnp.float32)

    pltpu.sync_copy(z_v, denom_sh.at[pl.ds(si * 2560, 2560)])

    pltpu.sync_copy(asrc_hbm, asrc_v)
    pltpu.sync_copy(adst_hbm, adst_v)
    pltpu.sync_copy(l_hbm, l_v)
    plsc.subcore_barrier()

    lane = lax.iota(jnp.int32, 16)
    h_pat = lane & 3
    e_pat = lane >> 2
    lvec = l_v[...]

    def chunk_base(k):
        return wid * EPW + jnp.minimum(k, NCH1) * CH1

    def in_descs(k, b):
        base = chunk_base(k)
        return (
            pltpu.make_async_copy(src_hbm.at[pl.ds(base, CH1)],
                                  src[b], sin[b]),
            pltpu.make_async_copy(dst_hbm.at[pl.ds(base, CH1)],
                                  dst[b], sin[b]),
        )

    def wb_desc(k, b):
        return pltpu.make_async_copy(
            ex[b], ex_hbm.at[pl.ds(chunk_base(k) * 4, CH1 * HEADS)], swb[b])

    for d in in_descs(0, 0):
        d.start()
    for d in in_descs(1, 1):
        d.start()

    @pl.loop(0, NCH1 + 1, step=2)
    def _(k0):
        for b in (0, 1):
            k = k0 + b
            for d in in_descs(k, b):
                d.wait()

            @pl.when(k >= 2)
            def _():
                wb_desc(k - 2, b).wait()

            @plsc.parallel_loop(0, CH1 * HEADS // 16, unroll=2)
            def _(g):
                e_vec = g * 4 + e_pat
                s_idx = plsc.load_gather(src[b], [e_vec])
                d_idx = plsc.load_gather(dst[b], [e_vec])
                a_s = plsc.load_gather(asrc_v, [s_idx * 4 + h_pat])
                a_d = plsc.load_gather(adst_v, [d_idx * 4 + h_pat])
                al = a_s + a_d
                al = jnp.maximum(al, al * 0.2)
                ex[b][pl.ds(g * 16, 16)] = jnp.exp(al - lvec)

            wb_desc(k, b).start()

            @pl.when(k < NCH1)
            def _():
                @pl.loop(0, CH1 * HEADS // 128)
                def _(s):
                    @plsc.parallel_loop(0, 8, unroll=2)
                    def _(g):
                        e_vec = s * 32 + g * 4 + e_pat
                        d_idx = plsc.load_gather(dst[b], [e_vec])
                        idx_v[pl.ds(g * 16, 16)] = d_idx * 4 + h_pat

                    pltpu.sync_copy(ex[b].at[pl.ds(s * 128, 128)],
                                    denom_sh.at[idx_v], add=True)

            for d in in_descs(k + 2, b):
                d.start()

    for d in in_descs(NCH1 + 2, 0):
        d.wait()
    for d in in_descs(NCH1 + 3, 1):
        d.wait()
    wb_desc(NCH1 - 1, 0).wait()
    wb_desc(NCH1, 1).wait()

    plsc.subcore_barrier()
    pltpu.sync_copy(denom_sh.at[pl.ds(si * 2560, 2560)],
                    denom_hbm.at[ci, pl.ds(si * 2560, 2560)])


def _inv_kernel(d_ref, inv_ref):
    d = d_ref[...]
    inv_ref[...] = 0.25 / (d[0] + d[1] + 1e-16)


def _inv_denom(denom):
    d3 = denom.reshape(NCORES, NP * HEADS // 128, 128)
    out = pl.pallas_call(
        _inv_kernel,
        out_shape=jax.ShapeDtypeStruct((NP * HEADS // 128, 128), jnp.float32),
    )(d3)
    return out.reshape(NP * HEADS)


@functools.partial(
    pl.kernel,
    mesh=_SC_MESH,
    out_type=jax.ShapeDtypeStruct((E * HEADS + E4PAD,), jnp.float32),
    scratch_types=[
        pltpu.VMEM((NP * HEADS,), jnp.float32),
        pltpu.VMEM((CH1,), jnp.int32),
        pltpu.VMEM((CH1,), jnp.int32),
        pltpu.VMEM((CH1 * HEADS,), jnp.float32),
        pltpu.VMEM((CH1 * HEADS,), jnp.float32),
        pltpu.SemaphoreType.DMA,
        pltpu.SemaphoreType.DMA,
        pltpu.SemaphoreType.DMA,
        pltpu.SemaphoreType.DMA,
    ],
    compiler_params=_SC_PARAMS,
)
def _sc_coef(invd_hbm, ex_hbm, dst_hbm, coef_hbm,
             inv_v, dst_a, dst_b, ex_a, ex_b, si_a, si_b, sw_a, sw_b):
    ci = lax.axis_index("c")
    si = lax.axis_index("s")
    wid = si * NCORES + ci

    dst = (dst_a, dst_b)
    ex = (ex_a, ex_b)
    sin = (si_a, si_b)
    swb = (sw_a, sw_b)

    NCH1 = EPW // CH1

    pltpu.sync_copy(invd_hbm, inv_v)

    lane = lax.iota(jnp.int32, 16)
    h_pat = lane & 3
    e_pat = lane >> 2

    def chunk_base(k):
        return wid * EPW + jnp.minimum(k, NCH1) * CH1

    def in_descs(k, b):
        base = chunk_base(k)
        return (
            pltpu.make_async_copy(dst_hbm.at[pl.ds(base, CH1)],
                                  dst[b], sin[b]),
            pltpu.make_async_copy(ex_hbm.at[pl.ds(base * 4, CH1 * HEADS)],
                                  ex[b], sin[b]),
        )

    def wb_desc(k, b):
        return pltpu.make_async_copy(
            ex[b], coef_hbm.at[pl.ds(chunk_base(k) * 4, CH1 * HEADS)], swb[b])

    for d in in_descs(0, 0):
        d.start()
    for d in in_descs(1, 1):
        d.start()

    @pl.loop(0, NCH1 + 1, step=2)
    def _(k0):
        for b in (0, 1):
            k = k0 + b
            for d in in_descs(k, b):
                d.wait()

            @pl.when(k >= 2)
            def _():
                wb_desc(k - 2, b).wait()

            @plsc.parallel_loop(0, CH1 * HEADS // 16, unroll=2)
            def _(g):
                e_vec = g * 4 + e_pat
                d_idx = plsc.load_gather(dst[b], [e_vec])
                iv = plsc.load_gather(inv_v, [d_idx * 4 + h_pat])
                ex[b][pl.ds(g * 16, 16)] = ex[b][pl.ds(g * 16, 16)] * iv

            wb_desc(k, b).start()
            for d in in_descs(k + 2, b):
                d.start()

    for d in in_descs(NCH1 + 2, 0):
        d.wait()
    for d in in_descs(NCH1 + 3, 1):
        d.wait()
    wb_desc(NCH1 - 1, 0).wait()
    wb_desc(NCH1, 1).wait()


@functools.partial(
    pl.kernel,
    mesh=_SC_MESH,
    out_type=jax.ShapeDtypeStruct((NCORES, NP, HID), jnp.float32),
    scratch_types=[
        pltpu.VMEM((CH2, HEADS * HID // 2), jnp.int32),
        pltpu.VMEM((CH2, HEADS * HID // 2), jnp.int32),
        pltpu.VMEM((CH2, HID), jnp.float32),
        pltpu.VMEM((CH2 * HEADS,), jnp.float32),
        pltpu.VMEM((CH2 * HEADS,), jnp.float32),
        pltpu.VMEM((CH2,), jnp.int32),
        pltpu.VMEM((CH2,), jnp.int32),
        pltpu.VMEM((CH2,), jnp.int32),
        pltpu.VMEM((CH2,), jnp.int32),
        pltpu.VMEM((CH2,), jnp.int32),
        pltpu.VMEM_SHARED((NP, HID), jnp.float32),
        pltpu.SemaphoreType.DMA,
        pltpu.SemaphoreType.DMA,
        pltpu.SemaphoreType.DMA,
        pltpu.SemaphoreType.DMA,
        pltpu.SemaphoreType.DMA,
    ],
    compiler_params=_SC_PARAMS,
)
def _sc_pass2(xs_hbm, coef_hbm, src_hbm, dst_hbm,
              out_hbm,
              rows_a, rows_b, m_v, coef_a, coef_b, src_a, src_b,
              dst_a, dst_b, dsc_v,
              out_sh, sg_a, sg_b, s1_a, s1_b, sc_v):
    ci = lax.axis_index("c")
    si = lax.axis_index("s")
    wid = si * NCORES + ci

    rows = (rows_a, rows_b)
    coef = (coef_a, coef_b)
    src = (src_a, src_b)
    dst = (dst_a, dst_b)
    sg = (sg_a, sg_b)
    s1 = (s1_a, s1_b)

    @pl.loop(0, CH2)
    def _(r):
        for c in range(HID // 16):
            m_v[r, pl.ds(c * 16, 16)] = jnp.zeros((16,), jnp.float32)

    @pl.loop(0, 640 // CH2)
    def _(j):
        pltpu.sync_copy(m_v, out_sh.at[pl.ds(si * 640 + j * CH2, CH2)])

    plsc.subcore_barrier()

    def stage1_descs(k, b):
        base = wid * EPW + jnp.minimum(k, NCH2) * CH2
        return (
            pltpu.make_async_copy(src_hbm.at[pl.ds(base, CH2)], src[b], s1[b]),
            pltpu.make_async_copy(dst_hbm.at[pl.ds(base, CH2)], dst[b], s1[b]),
            pltpu.make_async_copy(coef_hbm.at[pl.ds(base * 4, CH2 * HEADS)],
                                  coef[b], s1[b]),
        )

    def issue_stage1(k, b):
        for d in stage1_descs(k, b):
            d.start()

    def wait_stage1(k, b):
        for d in stage1_descs(k, b):
            d.wait()

    issue_stage1(0, 0)
    issue_stage1(1, 1)
    wait_stage1(0, 0)
    pltpu.make_async_copy(xs_hbm.at[src[0]], rows[0], sg[0]).start()

    @pl.loop(0, NIT2, step=2)
    def _(k0):
        for b in (0, 1):
            ob = 1 - b
            k = k0 + b
            pltpu.make_async_copy(xs_hbm.at[src[b]], rows[b], sg[b]).wait()
            wait_stage1(k + 1, ob)
            pltpu.make_async_copy(xs_hbm.at[src[ob]], rows[ob], sg[ob]).start()

            @pl.when(k >= 1)
            def _():
                pltpu.make_async_copy(m_v, out_sh.at[dsc_v], sc_v).wait()

            for j in range(CH2 // 16):
                dsc_v[pl.ds(j * 16, 16)] = dst[b][pl.ds(j * 16, 16)]

            mask0 = jnp.clip((EPW - k * CH2) * HEADS, 0, CH2 * HEADS)

            @pl.loop(mask0, CH2 * HEADS, step=16)
            def _(p):
                coef[b][pl.ds(p, 16)] = jnp.zeros((16,), jnp.float32)

            @plsc.parallel_loop(0, CH2, unroll=2)
            def _(e):
                cfs = [plsc.load_gather(
                           coef[b],
                           [jnp.full((16,), e * HEADS + h, jnp.int32)])
                       for h in range(HEADS)]
                for c in range(HID // 32):
                    acc_lo = None
                    acc_hi = None
                    for h in range(HEADS):
                        xi = rows[b][e, pl.ds(h * (HID // 2) + c * 16, 16)]
                        x32 = plsc.bitcast(xi, jnp.bfloat16)
                        lo, hi = plsc.unpack(
                            x32, format=plsc.PackFormat.INTERLEAVED)
                        if acc_lo is None:
                            acc_lo = lo * cfs[h]
                            acc_hi = hi * cfs[h]
                        else:
                            acc_lo = acc_lo + lo * cfs[h]
                            acc_hi = acc_hi + hi * cfs[h]
                    m_v[e, pl.ds(c * 32, 16)] = acc_lo
                    m_v[e, pl.ds(c * 32 + 16, 16)] = acc_hi

            pltpu.async_copy(m_v, out_sh.at[dsc_v], sc_v, add=True)
            issue_stage1(k + 2, b)

    pltpu.make_async_copy(xs_hbm.at[src[0]], rows[0], sg[0]).wait()
    wait_stage1(NIT2 + 1, 1)
    pltpu.make_async_copy(m_v, out_sh.at[dsc_v], sc_v).wait()

    plsc.subcore_barrier()
    pltpu.sync_copy(out_sh.at[pl.ds(si * 640, 640)],
                    out_hbm.at[ci, pl.ds(si * 640, 640)])


def _epilogue_kernel(o_ref, lin_ref, bc_ref, h_ref):
    o = o_ref[...]
    h_ref[...] = jax.nn.relu(o[0] + o[1] + bc_ref[...].reshape(1, HID)
                             + lin_ref[...])


def _epilogue(o_parts, lin, b_conv):
    bn = 2000
    return pl.pallas_call(
        _epilogue_kernel,
        grid=(N // bn,),
        in_specs=[
            pl.BlockSpec((NCORES, bn, HID), lambda i: (0, i, 0)),
            pl.BlockSpec((bn, HID), lambda i: (i, 0)),
            pl.BlockSpec((HID,), lambda i: (0,)),
        ],
        out_specs=pl.BlockSpec((bn, HID), lambda i: (i, 0)),
        out_shape=jax.ShapeDtypeStruct((N, HID), jnp.float32),
    )(o_parts, lin, b_conv)


def _gat_layer(h, src, dst, W_src, W_dst, att_src, att_dst, b_conv,
               W_lin, b_lin):
    xs, a_s, a_d, lin, ms, md = _dense_stage(
        h, W_src, W_dst, att_src, att_dst, W_lin, b_lin)
    l4 = jax.nn.leaky_relu(ms + md, negative_slope=0.2)
    l16 = jnp.tile(l4, (1, 4)).reshape(16)
    asrc_f = jnp.pad(a_s.reshape(-1), (0, (NP - N) * HEADS))
    adst_f = jnp.pad(a_d.reshape(-1), (0, (NP - N) * HEADS))
    ex, denom = _sc_pass1(asrc_f, adst_f, l16, src, dst)
    invd = _inv_denom(denom)
    coef = _sc_coef(invd, ex, dst)
    xsp = xs.reshape(N, 16, 2, 16).swapaxes(2, 3).reshape(N, 16, 16, 2)
    xs_i32 = jax.lax.bitcast_convert_type(xsp, jnp.int32).reshape(
        N, HEADS * HID // 2)
    o_parts = _sc_pass2(xs_i32, coef, src, dst)
    return _epilogue(o_parts, lin, b_conv)


def kernel(x, edge_index, W_src1, W_dst1, att_src1, att_dst1, b_conv1, W_lin1, b_lin1, W_src2, W_dst2, att_src2, att_dst2, b_conv2, W_lin2, b_lin2, W_src3, W_dst3, att_src3, att_dst3, b_conv3, W_lin3, b_lin3):
    src = jnp.pad(edge_index[0], (0, EPAD))
    dst = jnp.pad(edge_index[1], (0, EPAD))
    h = _gat_layer(x, src, dst, W_src1, W_dst1, att_src1, att_dst1, b_conv1,
                   W_lin1, b_lin1)
    h = _gat_layer(h, src, dst, W_src2, W_dst2, att_src2, att_dst2, b_conv2,
                   W_lin2, b_lin2)
    h = _gat_layer(h, src, dst, W_src3, W_dst3, att_src3, att_dst3, b_conv3,
                   W_lin3, b_lin3)
    return h
